# baseline (speedup 1.0000x reference)
import functools

import jax
import jax.numpy as jnp
import numpy as np
from jax.experimental import pallas as pl
from jax.experimental.pallas import tpu as pltpu
from jax.experimental.pallas import tpu_sc as plsc

P = 128
CB = 1024
H = 16
TOKENS = 16384
TB = 1024
GRID = TOKENS // TB

_ang = (2.0 * np.pi / P) * np.outer(np.arange(P), np.arange(P))
_DFT_COS = np.cos(_ang).astype(np.float32)
_DFT_SIN = np.sin(_ang).astype(np.float32)


def _encode_block(x_ref, c_ref, s_ref,
                  w0, b0, w1, b1, w2, b2,
                  r0w1, r0b1, r0w2, r0b2,
                  r1w1, r1b1, r1w2, r1b2,
                  scw, scb, wt_ref,
                  idx_ref, loss_ref):
    i = pl.program_id(0)
    xb = x_ref[...]
    f32 = jnp.float32
    doth = functools.partial(jnp.dot, preferred_element_type=f32,
                             precision=jax.lax.Precision.HIGHEST)
    dot = lambda a, b: jnp.dot(a.astype(jnp.bfloat16), b.astype(jnp.bfloat16),
                               preferred_element_type=f32)

    re = doth(xb, c_ref[...])
    im = doth(xb, s_ref[...])
    xf = jnp.sqrt(re * re + im * im)

    h = jax.nn.relu(dot(xf, w0[...]) + b0[...])
    h = jax.nn.relu(dot(h, w1[...]) + b1[...])
    h = dot(h, w2[...]) + b2[...]
    for rw1, rb1, rw2, rb2 in ((r0w1, r0b1, r0w2, r0b2),
                               (r1w1, r1b1, r1w2, r1b2)):
        m = jax.nn.relu(h)
        m = dot(m, rw1[...]) + rb1[...]
        m = jax.nn.relu(m)
        m = dot(m, rw2[...]) + rb2[...]
        h = h + m
    h = jax.nn.relu(h)
    z = h + (dot(xf, scw[...]) + scb[...])

    n = jnp.sqrt(jnp.sum(z * z, axis=-1, keepdims=True))
    z = z / jnp.maximum(n, 1e-12)

    wt = wt_ref[...]
    s = jnp.sum(z * z, axis=1, keepdims=True)
    w2sum = jnp.sum(wt * wt, axis=0)
    d = s + w2sum[None, :] - 2.0 * dot(z, wt)

    dmin = jnp.min(d, axis=1)
    ids = jax.lax.broadcasted_iota(jnp.int32, d.shape, 1)
    cand = jnp.where(d == dmin[:, None], ids, jnp.int32(2**30))
    idx = jnp.min(cand, axis=1)
    idx_ref[0, 0, :] = idx

    dex = s + w2sum[None, :] - 2.0 * doth(z, wt)
    dsel = jnp.sum(jnp.where(ids == idx[:, None], dex, 0.0), axis=1)
    prev = jnp.where(i == 0, jnp.zeros((1, 1), f32), loss_ref[...])
    loss_ref[...] = prev + jnp.sum(dsel)


def _decode_table(w_ref,
                  r0w1, r0b1, r0w2, r0b2,
                  r1w1, r1b1, r1w2, r1b2,
                  w0, b0, w1, b1, w2, b2,
                  scw, scb, out_ref):
    f32 = jnp.float32
    dot = lambda a, b: jnp.dot(a.astype(jnp.bfloat16), b.astype(jnp.bfloat16),
                               preferred_element_type=f32)
    x = w_ref[...]
    h = x
    for rw1, rb1, rw2, rb2 in ((r0w1, r0b1, r0w2, r0b2),
                               (r1w1, r1b1, r1w2, r1b2)):
        m = jax.nn.relu(h)
        m = dot(m, rw1[...]) + rb1[...]
        m = jax.nn.relu(m)
        m = dot(m, rw2[...]) + rb2[...]
        h = h + m
    h = jax.nn.relu(dot(h, w0[...]) + b0[...])
    h = jax.nn.relu(dot(h, w1[...]) + b1[...])
    h = dot(h, w2[...]) + b2[...]
    out_ref[...] = h + (dot(x, scw[...]) + scb[...])


def _sc_gather(table, idx_flat):
    n = idx_flat.shape[0]
    win = 128
    mesh = plsc.VectorSubcoreMesh(core_axis_name="c", subcore_axis_name="s")
    idx2 = idx_flat.reshape(1, n)

    @pl.kernel(out_type=jax.ShapeDtypeStruct((n, P), jnp.float32), mesh=mesh)
    def gk(tab_hbm, i_hbm, o_hbm):
        def body(i_vmem, o_vmem):
            pltpu.sync_copy(tab_hbm.at[i_vmem.at[0]], o_vmem)

        pltpu.emit_pipeline(
            body,
            grid=(n // win,),
            in_specs=[pl.BlockSpec((1, win), index_map=lambda i: (0, i))],
            out_specs=[pl.BlockSpec((win, P), index_map=lambda i: (i, 0))],
            core_axis_name=("c", "s"),
            dimension_semantics=(pltpu.PARALLEL,),
        )(i_hbm, o_hbm)

    return gk(table, idx2)


def kernel(x, params):
    B, V, L = x.shape
    xp = x.reshape(B * V * (L // P), P)
    p = params
    f32 = jnp.float32

    def row(v):
        return v.reshape(1, -1)

    C = jnp.asarray(_DFT_COS)
    S = jnp.asarray(_DFT_SIN)
    wt = p['codebook'].T

    enc_in = (
        xp, C, S,
        p['e_w0'], row(p['e_b0']), p['e_w1'], row(p['e_b1']),
        p['e_w2'], row(p['e_b2']),
        p['e_r0_w1'], row(p['e_r0_b1']), p['e_r0_w2'], row(p['e_r0_b2']),
        p['e_r1_w1'], row(p['e_r1_b1']), p['e_r1_w2'], row(p['e_r1_b2']),
        p['e_sc_w'], row(p['e_sc_b']), wt,
    )
    full = lambda a: pl.BlockSpec(a.shape, lambda i: (0,) * a.ndim)
    enc_specs = [pl.BlockSpec((TB, P), lambda i: (i, 0))]
    enc_specs += [full(a) for a in enc_in[1:]]

    idx3, loss_sum = pl.pallas_call(
        _encode_block,
        grid=(GRID,),
        in_specs=enc_specs,
        out_specs=[
            pl.BlockSpec((1, 1, TB), lambda i: (i, 0, 0)),
            pl.BlockSpec((1, 1), lambda i: (0, 0)),
        ],
        out_shape=[
            jax.ShapeDtypeStruct((GRID, 1, TB), jnp.int32),
            jax.ShapeDtypeStruct((1, 1), f32),
        ],
    )(*enc_in)
    idx = idx3.reshape(TOKENS)
    loss = (loss_sum[0, 0] / (TOKENS * H)).astype(f32)

    dec_in = (
        p['codebook'],
        p['d_r0_w1'], row(p['d_r0_b1']), p['d_r0_w2'], row(p['d_r0_b2']),
        p['d_r1_w1'], row(p['d_r1_b1']), p['d_r1_w2'], row(p['d_r1_b2']),
        p['d_w0'], row(p['d_b0']), p['d_w1'], row(p['d_b1']),
        p['d_w2'], row(p['d_b2']),
        p['d_sc_w'], row(p['d_sc_b']),
    )
    full0 = lambda a: pl.BlockSpec(a.shape, lambda: (0,) * a.ndim)
    table = pl.pallas_call(
        _decode_table,
        in_specs=[full0(a) for a in dec_in],
        out_specs=pl.BlockSpec((CB, P), lambda: (0, 0)),
        out_shape=jax.ShapeDtypeStruct((CB, P), f32),
    )(*dec_in)

    out = _sc_gather(table, idx).reshape(B * V, L // P, P)
    return out, loss, idx


# speedup vs baseline: 2.5401x; 2.5401x over previous
import functools

import jax
import jax.numpy as jnp
import numpy as np
from jax.experimental import pallas as pl

P = 128
CB = 1024
H = 16
TOKENS = 16384
TB = 1024
GRID = TOKENS // TB

_ang = (2.0 * np.pi / P) * np.outer(np.arange(P), np.arange(P))
_DFT_COS = np.cos(_ang).astype(np.float32)
_DFT_SIN = np.sin(_ang).astype(np.float32)


def _encode_block(x_ref, c_ref, s_ref,
                  w0, b0, w1, b1, w2, b2,
                  r0w1, r0b1, r0w2, r0b2,
                  r1w1, r1b1, r1w2, r1b2,
                  scw, scb, wt_ref, tabhi_ref, tablo_ref,
                  idx_ref, loss_ref, out_ref):
    i = pl.program_id(0)
    xb = x_ref[...]
    f32 = jnp.float32
    doth = functools.partial(jnp.dot, preferred_element_type=f32,
                             precision=jax.lax.Precision.HIGHEST)
    dot = lambda a, b: jnp.dot(a.astype(jnp.bfloat16), b.astype(jnp.bfloat16),
                               preferred_element_type=f32)

    re = doth(xb, c_ref[...])
    im = doth(xb, s_ref[...])
    xf = jnp.sqrt(re * re + im * im)

    h = jax.nn.relu(dot(xf, w0[...]) + b0[...])
    h = jax.nn.relu(dot(h, w1[...]) + b1[...])
    h = dot(h, w2[...]) + b2[...]
    for rw1, rb1, rw2, rb2 in ((r0w1, r0b1, r0w2, r0b2),
                               (r1w1, r1b1, r1w2, r1b2)):
        m = jax.nn.relu(h)
        m = dot(m, rw1[...]) + rb1[...]
        m = jax.nn.relu(m)
        m = dot(m, rw2[...]) + rb2[...]
        h = h + m
    h = jax.nn.relu(h)
    z = h + (dot(xf, scw[...]) + scb[...])

    n = jnp.sqrt(jnp.sum(z * z, axis=-1, keepdims=True))
    z = z / jnp.maximum(n, 1e-12)

    wt = wt_ref[...]
    s = jnp.sum(z * z, axis=1, keepdims=True)
    w2sum = jnp.sum(wt * wt, axis=0)
    d = s + w2sum[None, :] - 2.0 * dot(z, wt)

    dmin = jnp.min(d, axis=1)
    ids = jax.lax.broadcasted_iota(jnp.int32, d.shape, 1)
    cand = jnp.where(d == dmin[:, None], ids, jnp.int32(2**30))
    idx = jnp.min(cand, axis=1)
    idx_ref[0, 0, :] = idx

    prev = jnp.where(i == 0, jnp.zeros((1, 1), f32), loss_ref[...])
    loss_ref[...] = prev + jnp.sum(dmin)

    onehot = (ids == idx[:, None]).astype(jnp.bfloat16)
    out_ref[...] = (
        jnp.dot(onehot, tabhi_ref[...], preferred_element_type=f32)
        + jnp.dot(onehot, tablo_ref[...], preferred_element_type=f32))


def _decode_table(w_ref,
                  r0w1, r0b1, r0w2, r0b2,
                  r1w1, r1b1, r1w2, r1b2,
                  w0, b0, w1, b1, w2, b2,
                  scw, scb, hi_ref, lo_ref):
    f32 = jnp.float32
    dot = lambda a, b: jnp.dot(a.astype(jnp.bfloat16), b.astype(jnp.bfloat16),
                               preferred_element_type=f32)
    x = w_ref[...]
    h = x
    for rw1, rb1, rw2, rb2 in ((r0w1, r0b1, r0w2, r0b2),
                               (r1w1, r1b1, r1w2, r1b2)):
        m = jax.nn.relu(h)
        m = dot(m, rw1[...]) + rb1[...]
        m = jax.nn.relu(m)
        m = dot(m, rw2[...]) + rb2[...]
        h = h + m
    h = jax.nn.relu(dot(h, w0[...]) + b0[...])
    h = jax.nn.relu(dot(h, w1[...]) + b1[...])
    h = dot(h, w2[...]) + b2[...]
    tab = h + (dot(x, scw[...]) + scb[...])
    hi = tab.astype(jnp.bfloat16)
    hi_ref[...] = hi
    lo_ref[...] = (tab - hi.astype(f32)).astype(jnp.bfloat16)


def kernel(x, params):
    B, V, L = x.shape
    xp = x.reshape(B * V * (L // P), P)
    p = params
    f32 = jnp.float32

    def row(v):
        return v.reshape(1, -1)

    C = jnp.asarray(_DFT_COS)
    S = jnp.asarray(_DFT_SIN)
    wt = p['codebook'].T

    dec_in = (
        p['codebook'],
        p['d_r0_w1'], row(p['d_r0_b1']), p['d_r0_w2'], row(p['d_r0_b2']),
        p['d_r1_w1'], row(p['d_r1_b1']), p['d_r1_w2'], row(p['d_r1_b2']),
        p['d_w0'], row(p['d_b0']), p['d_w1'], row(p['d_b1']),
        p['d_w2'], row(p['d_b2']),
        p['d_sc_w'], row(p['d_sc_b']),
    )
    full0 = lambda a: pl.BlockSpec(a.shape, lambda: (0,) * a.ndim)
    tab_hi, tab_lo = pl.pallas_call(
        _decode_table,
        in_specs=[full0(a) for a in dec_in],
        out_specs=[pl.BlockSpec((CB, P), lambda: (0, 0)),
                   pl.BlockSpec((CB, P), lambda: (0, 0))],
        out_shape=[jax.ShapeDtypeStruct((CB, P), jnp.bfloat16),
                   jax.ShapeDtypeStruct((CB, P), jnp.bfloat16)],
    )(*dec_in)

    enc_in = (
        xp, C, S,
        p['e_w0'], row(p['e_b0']), p['e_w1'], row(p['e_b1']),
        p['e_w2'], row(p['e_b2']),
        p['e_r0_w1'], row(p['e_r0_b1']), p['e_r0_w2'], row(p['e_r0_b2']),
        p['e_r1_w1'], row(p['e_r1_b1']), p['e_r1_w2'], row(p['e_r1_b2']),
        p['e_sc_w'], row(p['e_sc_b']), wt, tab_hi, tab_lo,
    )
    full = lambda a: pl.BlockSpec(a.shape, lambda i: (0,) * a.ndim)
    enc_specs = [pl.BlockSpec((TB, P), lambda i: (i, 0))]
    enc_specs += [full(a) for a in enc_in[1:]]

    idx3, loss_sum, out = pl.pallas_call(
        _encode_block,
        grid=(GRID,),
        in_specs=enc_specs,
        out_specs=[
            pl.BlockSpec((1, 1, TB), lambda i: (i, 0, 0)),
            pl.BlockSpec((1, 1), lambda i: (0, 0)),
            pl.BlockSpec((TB, P), lambda i: (i, 0)),
        ],
        out_shape=[
            jax.ShapeDtypeStruct((GRID, 1, TB), jnp.int32),
            jax.ShapeDtypeStruct((1, 1), f32),
            jax.ShapeDtypeStruct((TOKENS, P), f32),
        ],
    )(*enc_in)
    idx = idx3.reshape(TOKENS)
    loss = (loss_sum[0, 0] / (TOKENS * H)).astype(f32)
    out = out.reshape(B * V, L // P, P)
    return out, loss, idx


# speedup vs baseline: 2.7281x; 1.0740x over previous
import functools

import jax
import jax.numpy as jnp
import numpy as np
from jax.experimental import pallas as pl

P = 128
CB = 1024
H = 16
TOKENS = 16384
TB = 2048
GRID = TOKENS // TB

_ang = (2.0 * np.pi / P) * np.outer(np.arange(P), np.arange(P))
_DFT_COS = np.cos(_ang).astype(np.float32)
_DFT_SIN = np.sin(_ang).astype(np.float32)


def _encode_block(x_ref, c_ref, s_ref,
                  w0, b0, w1, b1, w2, b2,
                  r0w1, r0b1, r0w2, r0b2,
                  r1w1, r1b1, r1w2, r1b2,
                  scw, scb, wt_ref, tabhi_ref, tablo_ref,
                  idx_ref, loss_ref, out_ref):
    i = pl.program_id(0)
    xb = x_ref[...]
    f32 = jnp.float32
    doth = functools.partial(jnp.dot, preferred_element_type=f32,
                             precision=jax.lax.Precision.HIGHEST)
    dot = lambda a, b: jnp.dot(a.astype(jnp.bfloat16), b.astype(jnp.bfloat16),
                               preferred_element_type=f32)

    re = doth(xb, c_ref[...])
    im = doth(xb, s_ref[...])
    xf = jnp.sqrt(re * re + im * im)

    h = jax.nn.relu(dot(xf, w0[...]) + b0[...])
    h = jax.nn.relu(dot(h, w1[...]) + b1[...])
    h = dot(h, w2[...]) + b2[...]
    for rw1, rb1, rw2, rb2 in ((r0w1, r0b1, r0w2, r0b2),
                               (r1w1, r1b1, r1w2, r1b2)):
        m = jax.nn.relu(h)
        m = dot(m, rw1[...]) + rb1[...]
        m = jax.nn.relu(m)
        m = dot(m, rw2[...]) + rb2[...]
        h = h + m
    h = jax.nn.relu(h)
    z = h + (dot(xf, scw[...]) + scb[...])

    n = jnp.sqrt(jnp.sum(z * z, axis=-1, keepdims=True))
    z = z / jnp.maximum(n, 1e-12)

    wt = wt_ref[...]
    s = jnp.sum(z * z, axis=1, keepdims=True)
    w2sum = jnp.sum(wt * wt, axis=0)
    d = s + w2sum[None, :] - 2.0 * dot(z, wt)

    dmin = jnp.min(d, axis=1)
    ids = jax.lax.broadcasted_iota(jnp.int32, d.shape, 1)
    cand = jnp.where(d == dmin[:, None], ids, jnp.int32(2**30))
    idx = jnp.min(cand, axis=1)
    idx_ref[0, 0, :] = idx

    prev = jnp.where(i == 0, jnp.zeros((1, 1), f32), loss_ref[...])
    loss_ref[...] = prev + jnp.sum(dmin)

    onehot = (ids == idx[:, None]).astype(jnp.bfloat16)
    out_ref[...] = (
        jnp.dot(onehot, tabhi_ref[...], preferred_element_type=f32)
        + jnp.dot(onehot, tablo_ref[...], preferred_element_type=f32))


def _decode_table(w_ref,
                  r0w1, r0b1, r0w2, r0b2,
                  r1w1, r1b1, r1w2, r1b2,
                  w0, b0, w1, b1, w2, b2,
                  scw, scb, hi_ref, lo_ref):
    f32 = jnp.float32
    dot = lambda a, b: jnp.dot(a.astype(jnp.bfloat16), b.astype(jnp.bfloat16),
                               preferred_element_type=f32)
    x = w_ref[...]
    h = x
    for rw1, rb1, rw2, rb2 in ((r0w1, r0b1, r0w2, r0b2),
                               (r1w1, r1b1, r1w2, r1b2)):
        m = jax.nn.relu(h)
        m = dot(m, rw1[...]) + rb1[...]
        m = jax.nn.relu(m)
        m = dot(m, rw2[...]) + rb2[...]
        h = h + m
    h = jax.nn.relu(dot(h, w0[...]) + b0[...])
    h = jax.nn.relu(dot(h, w1[...]) + b1[...])
    h = dot(h, w2[...]) + b2[...]
    tab = h + (dot(x, scw[...]) + scb[...])
    hi = tab.astype(jnp.bfloat16)
    hi_ref[...] = hi
    lo_ref[...] = (tab - hi.astype(f32)).astype(jnp.bfloat16)


def kernel(x, params):
    B, V, L = x.shape
    xp = x.reshape(B * V * (L // P), P)
    p = params
    f32 = jnp.float32

    def row(v):
        return v.reshape(1, -1)

    C = jnp.asarray(_DFT_COS)
    S = jnp.asarray(_DFT_SIN)
    wt = p['codebook'].T

    dec_in = (
        p['codebook'],
        p['d_r0_w1'], row(p['d_r0_b1']), p['d_r0_w2'], row(p['d_r0_b2']),
        p['d_r1_w1'], row(p['d_r1_b1']), p['d_r1_w2'], row(p['d_r1_b2']),
        p['d_w0'], row(p['d_b0']), p['d_w1'], row(p['d_b1']),
        p['d_w2'], row(p['d_b2']),
        p['d_sc_w'], row(p['d_sc_b']),
    )
    full0 = lambda a: pl.BlockSpec(a.shape, lambda: (0,) * a.ndim)
    tab_hi, tab_lo = pl.pallas_call(
        _decode_table,
        in_specs=[full0(a) for a in dec_in],
        out_specs=[pl.BlockSpec((CB, P), lambda: (0, 0)),
                   pl.BlockSpec((CB, P), lambda: (0, 0))],
        out_shape=[jax.ShapeDtypeStruct((CB, P), jnp.bfloat16),
                   jax.ShapeDtypeStruct((CB, P), jnp.bfloat16)],
    )(*dec_in)

    enc_in = (
        xp, C, S,
        p['e_w0'], row(p['e_b0']), p['e_w1'], row(p['e_b1']),
        p['e_w2'], row(p['e_b2']),
        p['e_r0_w1'], row(p['e_r0_b1']), p['e_r0_w2'], row(p['e_r0_b2']),
        p['e_r1_w1'], row(p['e_r1_b1']), p['e_r1_w2'], row(p['e_r1_b2']),
        p['e_sc_w'], row(p['e_sc_b']), wt, tab_hi, tab_lo,
    )
    full = lambda a: pl.BlockSpec(a.shape, lambda i: (0,) * a.ndim)
    enc_specs = [pl.BlockSpec((TB, P), lambda i: (i, 0))]
    enc_specs += [full(a) for a in enc_in[1:]]

    idx3, loss_sum, out = pl.pallas_call(
        _encode_block,
        grid=(GRID,),
        in_specs=enc_specs,
        out_specs=[
            pl.BlockSpec((1, 1, TB), lambda i: (i, 0, 0)),
            pl.BlockSpec((1, 1), lambda i: (0, 0)),
            pl.BlockSpec((TB, P), lambda i: (i, 0)),
        ],
        out_shape=[
            jax.ShapeDtypeStruct((GRID, 1, TB), jnp.int32),
            jax.ShapeDtypeStruct((1, 1), f32),
            jax.ShapeDtypeStruct((TOKENS, P), f32),
        ],
    )(*enc_in)
    idx = idx3.reshape(TOKENS)
    loss = (loss_sum[0, 0] / (TOKENS * H)).astype(f32)
    out = out.reshape(B * V, L // P, P)
    return out, loss, idx


# speedup vs baseline: 2.7542x; 1.0096x over previous
import functools

import jax
import jax.numpy as jnp
import numpy as np
from jax.experimental import pallas as pl

P = 128
CB = 1024
H = 16
TOKENS = 16384
TB = 4096
GRID = TOKENS // TB

_ang = (2.0 * np.pi / P) * np.outer(np.arange(P), np.arange(P))
_DFT_COS = np.cos(_ang).astype(np.float32)
_DFT_SIN = np.sin(_ang).astype(np.float32)


def _encode_block(x_ref, c_ref, s_ref,
                  w0, b0, w1, b1, w2, b2,
                  r0w1, r0b1, r0w2, r0b2,
                  r1w1, r1b1, r1w2, r1b2,
                  scw, scb, wt_ref, tabhi_ref, tablo_ref,
                  idx_ref, loss_ref, out_ref):
    i = pl.program_id(0)
    xb = x_ref[...]
    f32 = jnp.float32
    doth = functools.partial(jnp.dot, preferred_element_type=f32,
                             precision=jax.lax.Precision.HIGHEST)
    dot = lambda a, b: jnp.dot(a.astype(jnp.bfloat16), b.astype(jnp.bfloat16),
                               preferred_element_type=f32)

    re = doth(xb, c_ref[...])
    im = doth(xb, s_ref[...])
    xf = jnp.sqrt(re * re + im * im)

    h = jax.nn.relu(dot(xf, w0[...]) + b0[...])
    h = jax.nn.relu(dot(h, w1[...]) + b1[...])
    h = dot(h, w2[...]) + b2[...]
    for rw1, rb1, rw2, rb2 in ((r0w1, r0b1, r0w2, r0b2),
                               (r1w1, r1b1, r1w2, r1b2)):
        m = jax.nn.relu(h)
        m = dot(m, rw1[...]) + rb1[...]
        m = jax.nn.relu(m)
        m = dot(m, rw2[...]) + rb2[...]
        h = h + m
    h = jax.nn.relu(h)
    z = h + (dot(xf, scw[...]) + scb[...])

    n = jnp.sqrt(jnp.sum(z * z, axis=-1, keepdims=True))
    z = z / jnp.maximum(n, 1e-12)

    wt = wt_ref[...]
    s = jnp.sum(z * z, axis=1, keepdims=True)
    w2sum = jnp.sum(wt * wt, axis=0)
    d = s + w2sum[None, :] - 2.0 * dot(z, wt)

    dmin = jnp.min(d, axis=1)
    ids = jax.lax.broadcasted_iota(jnp.int32, d.shape, 1)
    cand = jnp.where(d == dmin[:, None], ids, jnp.int32(2**30))
    idx = jnp.min(cand, axis=1)
    idx_ref[0, 0, :] = idx

    prev = jnp.where(i == 0, jnp.zeros((1, 1), f32), loss_ref[...])
    loss_ref[...] = prev + jnp.sum(dmin)

    onehot = (ids == idx[:, None]).astype(jnp.bfloat16)
    out_ref[...] = (
        jnp.dot(onehot, tabhi_ref[...], preferred_element_type=f32)
        + jnp.dot(onehot, tablo_ref[...], preferred_element_type=f32))


def _decode_table(w_ref,
                  r0w1, r0b1, r0w2, r0b2,
                  r1w1, r1b1, r1w2, r1b2,
                  w0, b0, w1, b1, w2, b2,
                  scw, scb, hi_ref, lo_ref):
    f32 = jnp.float32
    dot = lambda a, b: jnp.dot(a.astype(jnp.bfloat16), b.astype(jnp.bfloat16),
                               preferred_element_type=f32)
    x = w_ref[...]
    h = x
    for rw1, rb1, rw2, rb2 in ((r0w1, r0b1, r0w2, r0b2),
                               (r1w1, r1b1, r1w2, r1b2)):
        m = jax.nn.relu(h)
        m = dot(m, rw1[...]) + rb1[...]
        m = jax.nn.relu(m)
        m = dot(m, rw2[...]) + rb2[...]
        h = h + m
    h = jax.nn.relu(dot(h, w0[...]) + b0[...])
    h = jax.nn.relu(dot(h, w1[...]) + b1[...])
    h = dot(h, w2[...]) + b2[...]
    tab = h + (dot(x, scw[...]) + scb[...])
    hi = tab.astype(jnp.bfloat16)
    hi_ref[...] = hi
    lo_ref[...] = (tab - hi.astype(f32)).astype(jnp.bfloat16)


def kernel(x, params):
    B, V, L = x.shape
    xp = x.reshape(B * V * (L // P), P)
    p = params
    f32 = jnp.float32

    def row(v):
        return v.reshape(1, -1)

    C = jnp.asarray(_DFT_COS)
    S = jnp.asarray(_DFT_SIN)
    wt = p['codebook'].T

    dec_in = (
        p['codebook'],
        p['d_r0_w1'], row(p['d_r0_b1']), p['d_r0_w2'], row(p['d_r0_b2']),
        p['d_r1_w1'], row(p['d_r1_b1']), p['d_r1_w2'], row(p['d_r1_b2']),
        p['d_w0'], row(p['d_b0']), p['d_w1'], row(p['d_b1']),
        p['d_w2'], row(p['d_b2']),
        p['d_sc_w'], row(p['d_sc_b']),
    )
    full0 = lambda a: pl.BlockSpec(a.shape, lambda: (0,) * a.ndim)
    tab_hi, tab_lo = pl.pallas_call(
        _decode_table,
        in_specs=[full0(a) for a in dec_in],
        out_specs=[pl.BlockSpec((CB, P), lambda: (0, 0)),
                   pl.BlockSpec((CB, P), lambda: (0, 0))],
        out_shape=[jax.ShapeDtypeStruct((CB, P), jnp.bfloat16),
                   jax.ShapeDtypeStruct((CB, P), jnp.bfloat16)],
    )(*dec_in)

    enc_in = (
        xp, C, S,
        p['e_w0'], row(p['e_b0']), p['e_w1'], row(p['e_b1']),
        p['e_w2'], row(p['e_b2']),
        p['e_r0_w1'], row(p['e_r0_b1']), p['e_r0_w2'], row(p['e_r0_b2']),
        p['e_r1_w1'], row(p['e_r1_b1']), p['e_r1_w2'], row(p['e_r1_b2']),
        p['e_sc_w'], row(p['e_sc_b']), wt, tab_hi, tab_lo,
    )
    full = lambda a: pl.BlockSpec(a.shape, lambda i: (0,) * a.ndim)
    enc_specs = [pl.BlockSpec((TB, P), lambda i: (i, 0))]
    enc_specs += [full(a) for a in enc_in[1:]]

    idx3, loss_sum, out = pl.pallas_call(
        _encode_block,
        grid=(GRID,),
        in_specs=enc_specs,
        out_specs=[
            pl.BlockSpec((1, 1, TB), lambda i: (i, 0, 0)),
            pl.BlockSpec((1, 1), lambda i: (0, 0)),
            pl.BlockSpec((TB, P), lambda i: (i, 0)),
        ],
        out_shape=[
            jax.ShapeDtypeStruct((GRID, 1, TB), jnp.int32),
            jax.ShapeDtypeStruct((1, 1), f32),
            jax.ShapeDtypeStruct((TOKENS, P), f32),
        ],
    )(*enc_in)
    idx = idx3.reshape(TOKENS)
    loss = (loss_sum[0, 0] / (TOKENS * H)).astype(f32)
    out = out.reshape(B * V, L // P, P)
    return out, loss, idx


# speedup vs baseline: 2.9106x; 1.0568x over previous
import functools

import jax
import jax.numpy as jnp
import numpy as np
from jax.experimental import pallas as pl
from jax.experimental.pallas import tpu as pltpu

P = 128
CB = 1024
H = 16
TOKENS = 16384
TB = 4096
GRID = TOKENS // TB
IDX_W = 128

_ang = (2.0 * np.pi / P) * np.outer(np.arange(P), np.arange(P))
_DFT_COS = np.cos(_ang).astype(np.float32)
_DFT_SIN = np.sin(_ang).astype(np.float32)


def _fused_block(x_ref, c_ref, s_ref,
                 w0, b0, w1, b1, w2, b2,
                 r0w1, r0b1, r0w2, r0b2,
                 r1w1, r1b1, r1w2, r1b2,
                 scw, scb, wt_ref, wtb_ref, cb_ref,
                 dr0w1, dr0b1, dr0w2, dr0b2,
                 dr1w1, dr1b1, dr1w2, dr1b2,
                 dw0, db0, dw1, db1, dw2, db2,
                 dscw, dscb,
                 idx_ref, loss_ref, out_ref,
                 tabhi_ref, tablo_ref):
    i = pl.program_id(0)
    f32 = jnp.float32
    bf16 = jnp.bfloat16
    doth = functools.partial(jnp.dot, preferred_element_type=f32,
                             precision=jax.lax.Precision.HIGHEST)
    dot = lambda a, b: jnp.dot(a.astype(bf16), b.astype(bf16),
                               preferred_element_type=f32)

    @pl.when(i == 0)
    def _decode_table():
        xw = cb_ref[...]
        h = xw
        for rw1, rb1, rw2, rb2 in ((dr0w1, dr0b1, dr0w2, dr0b2),
                                   (dr1w1, dr1b1, dr1w2, dr1b2)):
            m = jax.nn.relu(h)
            m = dot(m, rw1[...]) + rb1[...]
            m = jax.nn.relu(m)
            m = dot(m, rw2[...]) + rb2[...]
            h = h + m
        h = jax.nn.relu(dot(h, dw0[...]) + db0[...])
        h = jax.nn.relu(dot(h, dw1[...]) + db1[...])
        h = dot(h, dw2[...]) + db2[...]
        tab = h + (dot(xw, dscw[...]) + dscb[...])
        hi = tab.astype(bf16)
        tabhi_ref[...] = hi
        tablo_ref[...] = (tab - hi.astype(f32)).astype(bf16)

    xb = x_ref[...]
    re = doth(xb, c_ref[...])
    im = doth(xb, s_ref[...])
    xf = jnp.sqrt(re * re + im * im)

    h = jax.nn.relu(dot(xf, w0[...]) + b0[...])
    h = jax.nn.relu(dot(h, w1[...]) + b1[...])
    h = dot(h, w2[...]) + b2[...]
    for rw1, rb1, rw2, rb2 in ((r0w1, r0b1, r0w2, r0b2),
                               (r1w1, r1b1, r1w2, r1b2)):
        m = jax.nn.relu(h)
        m = dot(m, rw1[...]) + rb1[...]
        m = jax.nn.relu(m)
        m = dot(m, rw2[...]) + rb2[...]
        h = h + m
    h = jax.nn.relu(h)
    z = h + (dot(xf, scw[...]) + scb[...])

    n = jnp.sqrt(jnp.sum(z * z, axis=-1, keepdims=True))
    z = z / jnp.maximum(n, 1e-12)

    wt = wt_ref[...]
    s = jnp.sum(z * z, axis=1, keepdims=True)
    w2sum = jnp.sum(wt * wt, axis=0)
    d = s + w2sum[None, :] - 2.0 * jnp.dot(z.astype(bf16), wtb_ref[...],
                                           preferred_element_type=f32)

    dmin = jnp.min(d, axis=1)
    ids = jax.lax.broadcasted_iota(jnp.int32, d.shape, 1)
    cand = jnp.where(d == dmin[:, None], ids, jnp.int32(2**30))
    idx = jnp.min(cand, axis=1)
    idx_ref[...] = idx.reshape(TB // IDX_W, IDX_W)

    prev = jnp.where(i == 0, jnp.zeros((1, 1), f32), loss_ref[...])
    loss_ref[...] = prev + jnp.sum(dmin) * (1.0 / (TOKENS * H))

    onehot = (ids == idx[:, None]).astype(bf16)
    out_ref[...] = (
        jnp.dot(onehot, tabhi_ref[...], preferred_element_type=f32)
        + jnp.dot(onehot, tablo_ref[...], preferred_element_type=f32))


def kernel(x, params):
    B, V, L = x.shape
    xp = x.reshape(B * V * (L // P), P)
    p = params
    f32 = jnp.float32

    def row(v):
        return v.reshape(1, -1)

    C = jnp.asarray(_DFT_COS)
    S = jnp.asarray(_DFT_SIN)
    wt = p['codebook'].T
    wtb = wt.astype(jnp.bfloat16)

    ins = (
        xp, C, S,
        p['e_w0'], row(p['e_b0']), p['e_w1'], row(p['e_b1']),
        p['e_w2'], row(p['e_b2']),
        p['e_r0_w1'], row(p['e_r0_b1']), p['e_r0_w2'], row(p['e_r0_b2']),
        p['e_r1_w1'], row(p['e_r1_b1']), p['e_r1_w2'], row(p['e_r1_b2']),
        p['e_sc_w'], row(p['e_sc_b']), wt, wtb, p['codebook'],
        p['d_r0_w1'], row(p['d_r0_b1']), p['d_r0_w2'], row(p['d_r0_b2']),
        p['d_r1_w1'], row(p['d_r1_b1']), p['d_r1_w2'], row(p['d_r1_b2']),
        p['d_w0'], row(p['d_b0']), p['d_w1'], row(p['d_b1']),
        p['d_w2'], row(p['d_b2']),
        p['d_sc_w'], row(p['d_sc_b']),
    )
    full = lambda a: pl.BlockSpec(a.shape, lambda i: (0,) * a.ndim)
    in_specs = [pl.BlockSpec((TB, P), lambda i: (i, 0))]
    in_specs += [full(a) for a in ins[1:]]

    idx2, loss_sum, out = pl.pallas_call(
        _fused_block,
        grid=(GRID,),
        in_specs=in_specs,
        out_specs=[
            pl.BlockSpec((TB // IDX_W, IDX_W), lambda i: (i, 0)),
            pl.BlockSpec((1, 1), lambda i: (0, 0)),
            pl.BlockSpec((TB, P), lambda i: (i, 0)),
        ],
        out_shape=[
            jax.ShapeDtypeStruct((TOKENS // IDX_W, IDX_W), jnp.int32),
            jax.ShapeDtypeStruct((1, 1), f32),
            jax.ShapeDtypeStruct((TOKENS, P), f32),
        ],
        scratch_shapes=[
            pltpu.VMEM((CB, P), jnp.bfloat16),
            pltpu.VMEM((CB, P), jnp.bfloat16),
        ],
    )(*ins)
    idx = idx2.reshape(TOKENS)
    loss = loss_sum.reshape(())
    out = out.reshape(B * V, L // P, P)
    return out, loss, idx
